# baseline (device time: 25065 ns/iter reference)
import jax
import jax.numpy as jnp
from jax import lax
from jax.experimental import pallas as pl
from jax.experimental.pallas import tpu as pltpu

N_DEV = 32
N_GROUPS = 4
G = N_DEV // N_GROUPS


def kernel(x, w_mat, scale_x, scale_w):
    m_global, k_per = x.shape
    k_global, n = w_mat.shape
    m_per = m_global // N_DEV
    kb = G * k_per

    def body(x_ref, w_hbm, sx_ref, sw_ref, out_ref,
             xg_ref, w_ref, send_sems, recv_sems, w_sems):
        my = lax.axis_index("i")
        my_g = my // G

        barrier = pltpu.get_barrier_semaphore()
        for k in range(1, N_DEV):
            peer = (my + k) % N_DEV
            pl.semaphore_signal(
                barrier, inc=1,
                device_id=(peer,), device_id_type=pl.DeviceIdType.MESH,
            )
        pl.semaphore_wait(barrier, N_DEV - 1)

        for p in range(N_GROUPS):
            pltpu.make_async_copy(
                w_hbm.at[pl.ds(p * kb, kb), :],
                w_ref.at[pl.ds(p * kb, kb), :],
                w_sems.at[p],
            ).start()

        sends = []
        for jt in range(1, N_GROUPS + 1):
            tg = (my_g + jt) % N_GROUPS
            for u in range(G):
                peer = tg * G + u
                rdma = pltpu.make_async_remote_copy(
                    src_ref=x_ref.at[pl.ds(peer * m_per, m_per), :],
                    dst_ref=xg_ref.at[:, pl.ds(my * k_per, k_per)],
                    send_sem=send_sems.at[peer],
                    recv_sem=recv_sems.at[my],
                    device_id=(peer,),
                    device_id_type=pl.DeviceIdType.MESH,
                )
                rdma.start()
                sends.append(rdma)

        acc = None
        for t in range(N_GROUPS):
            g = (my_g - 1 - t) % N_GROUPS
            for u in range(G):
                s = g * G + u
                recv = pltpu.make_async_remote_copy(
                    src_ref=x_ref.at[pl.ds(s * m_per, m_per), :],
                    dst_ref=xg_ref.at[:, pl.ds(s * k_per, k_per)],
                    send_sem=send_sems.at[s],
                    recv_sem=recv_sems.at[s],
                    device_id=(s,),
                    device_id_type=pl.DeviceIdType.MESH,
                )
                recv.wait_recv()
            pltpu.make_async_copy(
                w_hbm.at[pl.ds(g * kb, kb), :],
                w_ref.at[pl.ds(g * kb, kb), :],
                w_sems.at[g],
            ).wait()
            part = lax.dot_general(
                xg_ref[:, pl.ds(g * kb, kb)],
                w_ref[pl.ds(g * kb, kb), :],
                (((1,), (0,)), ((), ())),
                preferred_element_type=jnp.int32,
            )
            acc = part if acc is None else acc + part

        out_ref[:, :] = acc.astype(jnp.float32) * (sx_ref[0] * sw_ref[0])

        for rdma in sends:
            rdma.wait_send()

    return pl.pallas_call(
        body,
        out_shape=jax.ShapeDtypeStruct((m_per, n), jnp.float32),
        in_specs=[
            pl.BlockSpec(memory_space=pltpu.VMEM),
            pl.BlockSpec(memory_space=pl.ANY),
            pl.BlockSpec(memory_space=pltpu.SMEM),
            pl.BlockSpec(memory_space=pltpu.SMEM),
        ],
        out_specs=pl.BlockSpec(memory_space=pltpu.VMEM),
        scratch_shapes=[
            pltpu.VMEM((m_per, k_global), jnp.int8),
            pltpu.VMEM((k_global, n), jnp.int8),
            pltpu.SemaphoreType.DMA((N_DEV,)),
            pltpu.SemaphoreType.DMA((N_DEV,)),
            pltpu.SemaphoreType.DMA((N_GROUPS,)),
        ],
        compiler_params=pltpu.CompilerParams(collective_id=0),
    )(x, w_mat, scale_x, scale_w)
